# Initial kernel scaffold; baseline (speedup 1.0000x reference)
import jax
import jax.numpy as jnp
from jax.experimental import pallas as pl


def kernel(x, edge_index, W1, a_src1, a_dst1, b1, W2, a_src2, a_dst2, b2, W3, a_src3, a_dst3, b3, W4, a_src4, a_dst4, b4, W5, a_src5, a_dst5, b5):
    raise NotImplementedError("write your pallas kernel here")


# speedup vs baseline: 70.3005x; 70.3005x over previous
import functools

import jax
import jax.numpy as jnp
from jax import lax
from jax.experimental import pallas as pl
from jax.experimental.pallas import tpu as pltpu
from jax.experimental.pallas import tpu_sc as plsc

NN = 10000
NPAD = 10240
EDGES = 320000
NCLS = 32
F32 = jnp.float32

NCORES = 2
NSUB = 16
NW = NCORES * NSUB
EPT = EDGES // NW
CH = 80
NCHUNK = EPT // CH
RPT = NPAD // NSUB

_GATHER_MODE = lax.GatherScatterMode.PROMISE_IN_BOUNDS


_GATHER_DNUMS = lax.GatherDimensionNumbers(
    offset_dims=(), collapsed_slice_dims=(0,), start_index_map=(0,))


def _take16(vec, idx):
    return lax.gather(vec, idx.reshape(16, 1), _GATHER_DNUMS, (1,),
                      mode=_GATHER_MODE)


def _iota16():
    return lax.iota(jnp.int32, 16)


_BM = 512
_GRID = NPAD // _BM


def _tc1_body(x_ref, w_ref, g_ref, gd_ref, hs_ref, ad_ref):
    h = jnp.dot(x_ref[...], w_ref[...], preferred_element_type=F32)
    hs_ref[...] = jnp.dot(h, g_ref[...], preferred_element_type=F32)
    ad_ref[...] = jnp.dot(h, gd_ref[...], preferred_element_type=F32)


def _tc1(xp, W, G, Gd):
    return pl.pallas_call(
        _tc1_body,
        grid=(_GRID,),
        in_specs=[
            pl.BlockSpec((_BM, xp.shape[1]), lambda i: (i, 0)),
            pl.BlockSpec(W.shape, lambda i: (0, 0)),
            pl.BlockSpec(G.shape, lambda i: (0, 0)),
            pl.BlockSpec(Gd.shape, lambda i: (0, 0)),
        ],
        out_specs=[
            pl.BlockSpec((_BM, 80), lambda i: (i, 0)),
            pl.BlockSpec((_BM, 16), lambda i: (i, 0)),
        ],
        out_shape=[
            jax.ShapeDtypeStruct((NPAD, 80), F32),
            jax.ShapeDtypeStruct((NPAD, 16), F32),
        ],
    )(xp, W, G, Gd)


def _act_from_parts(p_ref, b_ref, er_ref):
    psum = p_ref[0] + p_ref[1]
    num = psum[:, 0:64]
    den8 = psum[:, 64:72]
    den = jnp.dot(den8, er_ref[...], preferred_element_type=F32)
    act = num / (den + 1e-16) + b_ref[0][None, :]
    return jnp.maximum(act, 0.2 * act)


def _tcmid_body(p_ref, b_ref, er_ref, w_ref, g_ref, gd_ref, hs_ref, ad_ref):
    act = _act_from_parts(p_ref, b_ref, er_ref)
    h = jnp.dot(act, w_ref[...], preferred_element_type=F32)
    hs_ref[...] = jnp.dot(h, g_ref[...], preferred_element_type=F32)
    ad_ref[...] = jnp.dot(h, gd_ref[...], preferred_element_type=F32)


def _tcmid(p, b_cm, Erep, W_cm, G, Gd):
    return pl.pallas_call(
        _tcmid_body,
        grid=(_GRID,),
        in_specs=[
            pl.BlockSpec((2, _BM, 80), lambda i: (0, i, 0)),
            pl.BlockSpec((1, 64), lambda i: (0, 0)),
            pl.BlockSpec((8, 64), lambda i: (0, 0)),
            pl.BlockSpec((64, 64), lambda i: (0, 0)),
            pl.BlockSpec((64, 80), lambda i: (0, 0)),
            pl.BlockSpec((64, 16), lambda i: (0, 0)),
        ],
        out_specs=[
            pl.BlockSpec((_BM, 80), lambda i: (i, 0)),
            pl.BlockSpec((_BM, 16), lambda i: (i, 0)),
        ],
        out_shape=[
            jax.ShapeDtypeStruct((NPAD, 80), F32),
            jax.ShapeDtypeStruct((NPAD, 16), F32),
        ],
    )(p, b_cm, Erep, W_cm, G, Gd)


def _tcpre5_body(p_ref, b_ref, er_ref, w_ref, ga_ref, gd_ref,
                 hs_ref, ad_ref, as_ref):
    act = _act_from_parts(p_ref, b_ref, er_ref)
    h5 = jnp.dot(act, w_ref[...], preferred_element_type=F32)
    asd = jnp.dot(h5, ga_ref[...], preferred_element_type=F32)
    add = jnp.dot(h5, gd_ref[...], preferred_element_type=F32)
    hs_ref[:, 0:256] = h5
    hs_ref[:, 256:272] = asd
    as_ref[...] = asd
    ad_ref[...] = add


def _tcpre5(p, b_cm, Erep, W5_cm, G5a, G5d):
    return pl.pallas_call(
        _tcpre5_body,
        grid=(_GRID,),
        in_specs=[
            pl.BlockSpec((2, _BM, 80), lambda i: (0, i, 0)),
            pl.BlockSpec((1, 64), lambda i: (0, 0)),
            pl.BlockSpec((8, 64), lambda i: (0, 0)),
            pl.BlockSpec((64, 256), lambda i: (0, 0)),
            pl.BlockSpec((256, 16), lambda i: (0, 0)),
            pl.BlockSpec((256, 16), lambda i: (0, 0)),
        ],
        out_specs=[
            pl.BlockSpec((_BM, 272), lambda i: (i, 0)),
            pl.BlockSpec((_BM, 16), lambda i: (i, 0)),
            pl.BlockSpec((_BM, 16), lambda i: (i, 0)),
        ],
        out_shape=[
            jax.ShapeDtypeStruct((NPAD, 272), F32),
            jax.ShapeDtypeStruct((NPAD, 16), F32),
            jax.ShapeDtypeStruct((NPAD, 16), F32),
        ],
    )(p, b_cm, Erep, W5_cm, G5a, G5d)


def _tcmid5_body(ad_ref, da_ref, adr_ref):
    dsum = da_ref[0] + da_ref[1]
    rcp = 1.0 / (dsum + 1e-16)
    adr_ref[:, 0:8] = ad_ref[:, 0:8]
    adr_ref[:, 8:16] = rcp[:, 0:8]


def _tcmid5(ad5, dA):
    return pl.pallas_call(
        _tcmid5_body,
        grid=(_GRID,),
        in_specs=[
            pl.BlockSpec((_BM, 16), lambda i: (i, 0)),
            pl.BlockSpec((2, _BM, 16), lambda i: (0, i, 0)),
        ],
        out_specs=[pl.BlockSpec((_BM, 16), lambda i: (i, 0))],
        out_shape=[jax.ShapeDtypeStruct((NPAD, 16), F32)],
    )(ad5, dA)[0]


def _tcfinal_body(q_ref, b_ref, o_ref):
    logits = (q_ref[0] + q_ref[1]) * 0.125 + b_ref[0][None, :]
    m = jnp.max(logits, axis=1, keepdims=True)
    z = logits - m
    lse = jnp.log(jnp.sum(jnp.exp(z), axis=1, keepdims=True))
    o_ref[...] = z - lse


def _tcfinal(q, b5):
    return pl.pallas_call(
        _tcfinal_body,
        grid=(_GRID,),
        in_specs=[
            pl.BlockSpec((2, _BM, NCLS), lambda i: (0, i, 0)),
            pl.BlockSpec((1, NCLS), lambda i: (0, 0)),
        ],
        out_specs=[pl.BlockSpec((_BM, NCLS), lambda i: (i, 0))],
        out_shape=[jax.ShapeDtypeStruct((NPAD, NCLS), F32)],
    )(q, b5.reshape(1, NCLS))[0]


_MESH = plsc.VectorSubcoreMesh(core_axis_name="c", subcore_axis_name="s")
_SC_PARAMS = pltpu.CompilerParams(use_tc_tiling_on_sc=False)


def _sc_layer_body(hs_hbm, ad_hbm, srce_hbm, dste_hbm, z_hbm, out_hbm,
                   acc, src_v, dst_v, hsrows, adrows, crows, sem1, sem2):
    c = lax.axis_index("c")
    s = lax.axis_index("s")
    wid = s * NCORES + c
    lanes = _iota16()
    idxrep = lanes % 8

    pltpu.sync_copy(z_hbm, acc.at[pl.ds(s * RPT, RPT)])
    plsc.subcore_barrier()

    def chunk(k, carry):
        off = wid * EPT + k * CH
        pltpu.sync_copy(srce_hbm.at[pl.ds(off, CH)], src_v)
        pltpu.sync_copy(dste_hbm.at[pl.ds(off, CH)], dst_v)
        cp1 = pltpu.async_copy(hs_hbm.at[src_v], hsrows, sem1)
        cp2 = pltpu.async_copy(ad_hbm.at[dst_v], adrows, sem2)
        cp1.wait()
        cp2.wait()

        def edge(e, cc):
            a_s = hsrows[e, pl.ds(64, 16)]
            a_d = adrows[e, pl.ds(0, 16)]
            t = a_s + a_d
            t = jnp.maximum(t, 0.2 * t)
            ex = jnp.exp(t)
            w = _take16(ex, idxrep)
            crows[e, pl.ds(0, 16)] = hsrows[e, pl.ds(0, 16)] * w
            crows[e, pl.ds(16, 16)] = hsrows[e, pl.ds(16, 16)] * w
            crows[e, pl.ds(32, 16)] = hsrows[e, pl.ds(32, 16)] * w
            crows[e, pl.ds(48, 16)] = hsrows[e, pl.ds(48, 16)] * w
            crows[e, pl.ds(64, 16)] = jnp.where(lanes < 8, ex, 0.0)
            return cc

        lax.fori_loop(0, CH, edge, 0)
        pltpu.sync_copy(crows, acc.at[dst_v], add=True)
        return carry

    lax.fori_loop(0, NCHUNK, chunk, 0)
    plsc.subcore_barrier()
    pltpu.sync_copy(acc.at[pl.ds(s * RPT, RPT)],
                    out_hbm.at[c, pl.ds(s * RPT, RPT)])


def _sc_layer(hs, ad, srce, dste, z80):
    f = pl.kernel(
        _sc_layer_body,
        compiler_params=_SC_PARAMS,
        out_type=jax.ShapeDtypeStruct((2, NPAD, 80), F32),
        mesh=_MESH,
        scratch_types=[
            pltpu.VMEM_SHARED((NPAD, 80), F32),
            pltpu.VMEM((CH,), jnp.int32),
            pltpu.VMEM((CH,), jnp.int32),
            pltpu.VMEM((CH, 80), F32),
            pltpu.VMEM((CH, 16), F32),
            pltpu.VMEM((CH, 80), F32),
            pltpu.SemaphoreType.DMA,
            pltpu.SemaphoreType.DMA,
        ],
    )
    return f(hs, ad, srce, dste, z80)


def _sc_denom5_body(as_hbm, ad_hbm, srce_hbm, dste_hbm, z_hbm, out_hbm,
                    acc, src_v, dst_v, asrows, adrows, crows, sem1, sem2):
    c = lax.axis_index("c")
    s = lax.axis_index("s")
    wid = s * NCORES + c
    lanes8 = _iota16()

    pltpu.sync_copy(z_hbm, acc.at[pl.ds(s * RPT, RPT)])
    plsc.subcore_barrier()

    def chunk(k, carry):
        off = wid * EPT + k * CH
        pltpu.sync_copy(srce_hbm.at[pl.ds(off, CH)], src_v)
        pltpu.sync_copy(dste_hbm.at[pl.ds(off, CH)], dst_v)
        cp1 = pltpu.async_copy(as_hbm.at[src_v], asrows, sem1)
        cp2 = pltpu.async_copy(ad_hbm.at[dst_v], adrows, sem2)
        cp1.wait()
        cp2.wait()

        def edge(e, cc):
            t = asrows[e, pl.ds(0, 16)] + adrows[e, pl.ds(0, 16)]
            t = jnp.maximum(t, 0.2 * t)
            crows[e, pl.ds(0, 16)] = jnp.where(lanes8 < 8, jnp.exp(t), 0.0)
            return cc

        lax.fori_loop(0, CH, edge, 0)
        pltpu.sync_copy(crows, acc.at[dst_v], add=True)
        return carry

    lax.fori_loop(0, NCHUNK, chunk, 0)
    plsc.subcore_barrier()
    pltpu.sync_copy(acc.at[pl.ds(s * RPT, RPT)],
                    out_hbm.at[c, pl.ds(s * RPT, RPT)])


def _sc_denom5(as5, ad5, srce, dste, z16):
    f = pl.kernel(
        _sc_denom5_body,
        compiler_params=_SC_PARAMS,
        out_type=jax.ShapeDtypeStruct((2, NPAD, 16), F32),
        mesh=_MESH,
        scratch_types=[
            pltpu.VMEM_SHARED((NPAD, 16), F32),
            pltpu.VMEM((CH,), jnp.int32),
            pltpu.VMEM((CH,), jnp.int32),
            pltpu.VMEM((CH, 16), F32),
            pltpu.VMEM((CH, 16), F32),
            pltpu.VMEM((CH, 16), F32),
            pltpu.SemaphoreType.DMA,
            pltpu.SemaphoreType.DMA,
        ],
    )
    return f(as5, ad5, srce, dste, z16)


def _sc_aggr5_body(hs_hbm, adr_hbm, srce_hbm, dste_hbm, z_hbm, out_hbm,
                   acc, src_v, dst_v, hsrows, adrows, crows, sem1, sem2):
    c = lax.axis_index("c")
    s = lax.axis_index("s")
    wid = s * NCORES + c
    lanes = _iota16()
    idxrcp = lanes % 8 + 8

    pltpu.sync_copy(z_hbm, acc.at[pl.ds(s * RPT, RPT)])
    plsc.subcore_barrier()

    def chunk(k, carry):
        off = wid * EPT + k * CH
        pltpu.sync_copy(srce_hbm.at[pl.ds(off, CH)], src_v)
        pltpu.sync_copy(dste_hbm.at[pl.ds(off, CH)], dst_v)
        cp1 = pltpu.async_copy(hs_hbm.at[src_v], hsrows, sem1)
        cp2 = pltpu.async_copy(adr_hbm.at[dst_v], adrows, sem2)
        cp1.wait()
        cp2.wait()

        def edge(e, cc):
            a_s = hsrows[e, pl.ds(256, 16)]
            adr = adrows[e, pl.ds(0, 16)]
            t = a_s + adr
            t = jnp.maximum(t, 0.2 * t)
            ex = jnp.exp(t)
            w = ex * _take16(adr, idxrcp)
            acc0 = jnp.zeros((16,), F32)
            acc1 = jnp.zeros((16,), F32)
            for kk in range(16):
                wk = _take16(w, jnp.full((16,), kk // 2, jnp.int32))
                hk = hsrows[e, pl.ds(16 * kk, 16)]
                if kk % 2 == 0:
                    acc0 = acc0 + wk * hk
                else:
                    acc1 = acc1 + wk * hk
            crows[e, pl.ds(0, 16)] = acc0
            crows[e, pl.ds(16, 16)] = acc1
            return cc

        lax.fori_loop(0, CH, edge, 0)
        pltpu.sync_copy(crows, acc.at[dst_v], add=True)
        return carry

    lax.fori_loop(0, NCHUNK, chunk, 0)
    plsc.subcore_barrier()
    pltpu.sync_copy(acc.at[pl.ds(s * RPT, RPT)],
                    out_hbm.at[c, pl.ds(s * RPT, RPT)])


def _sc_aggr5(hs5, adr, srce, dste, z32):
    f = pl.kernel(
        _sc_aggr5_body,
        compiler_params=_SC_PARAMS,
        out_type=jax.ShapeDtypeStruct((2, NPAD, NCLS), F32),
        mesh=_MESH,
        scratch_types=[
            pltpu.VMEM_SHARED((NPAD, NCLS), F32),
            pltpu.VMEM((CH,), jnp.int32),
            pltpu.VMEM((CH,), jnp.int32),
            pltpu.VMEM((CH, 272), F32),
            pltpu.VMEM((CH, 16), F32),
            pltpu.VMEM((CH, NCLS), F32),
            pltpu.SemaphoreType.DMA,
            pltpu.SemaphoreType.DMA,
        ],
    )
    return f(hs5, adr, srce, dste, z32)


def _sel_mats(a_src, a_dst):
    hh, cc = a_src.shape
    eye = jnp.eye(hh, dtype=F32)
    s_src = jnp.einsum("hc,hk->hck", a_src, eye).reshape(hh * cc, hh)
    s_dst = jnp.einsum("hc,hk->hck", a_dst, eye).reshape(hh * cc, hh)
    return s_src, s_dst


def _perm64():
    ar = jnp.arange(64)
    perm = (ar % 8) * 8 + ar // 8
    return jnp.eye(64, dtype=F32)[perm]


def kernel(x, edge_index, W1, a_src1, a_dst1, b1, W2, a_src2, a_dst2, b2,
           W3, a_src3, a_dst3, b3, W4, a_src4, a_dst4, b4,
           W5, a_src5, a_dst5, b5):
    P = _perm64()
    Erep = jnp.tile(jnp.eye(8, dtype=F32), (1, 8))
    z8 = jnp.zeros((64, 8), F32)
    z80 = jnp.zeros((RPT, 80), F32)
    z16 = jnp.zeros((RPT, 16), F32)
    z32 = jnp.zeros((RPT, NCLS), F32)

    def build_G(a_src, a_dst):
        s_src, s_dst = _sel_mats(a_src, a_dst)
        G = jnp.concatenate([P, s_src, z8], axis=1)
        Gd = jnp.concatenate([s_dst, z8], axis=1)
        return G, Gd

    xp = jnp.pad(x, ((0, NPAD - NN), (0, 0)))
    ei = edge_index.astype(jnp.int32)
    srce, dste = ei[0], ei[1]

    G, Gd = build_G(a_src1, a_dst1)
    hs, ad = _tc1(xp, W1, G, Gd)
    p = _sc_layer(hs, ad, srce, dste, z80)

    for W, a_s_, a_d_, b_prev in (
        (W2, a_src2, a_dst2, b1),
        (W3, a_src3, a_dst3, b2),
        (W4, a_src4, a_dst4, b3),
    ):
        G, Gd = build_G(a_s_, a_d_)
        hs, ad = _tcmid(p, (b_prev @ P).reshape(1, 64), Erep, P @ W, G, Gd)
        p = _sc_layer(hs, ad, srce, dste, z80)

    s5_src, s5_dst = _sel_mats(a_src5, a_dst5)
    z58 = jnp.zeros((256, 8), F32)
    G5a = jnp.concatenate([s5_src, z58], axis=1)
    G5d = jnp.concatenate([s5_dst, z58], axis=1)
    hs5, ad5, as5 = _tcpre5(p, (b4 @ P).reshape(1, 64), Erep, P @ W5, G5a, G5d)
    dA = _sc_denom5(as5, ad5, srce, dste, z16)
    adr = _tcmid5(ad5, dA)
    q = _sc_aggr5(hs5, adr, srce, dste, z32)
    out = _tcfinal(q, b5)
    return out[:NN]


# speedup vs baseline: 78.8874x; 1.1221x over previous
import functools

import jax
import jax.numpy as jnp
from jax import lax
from jax.experimental import pallas as pl
from jax.experimental.pallas import tpu as pltpu
from jax.experimental.pallas import tpu_sc as plsc

NN = 10000
NPAD = 10240
EDGES = 320000
NCLS = 32
F32 = jnp.float32

NCORES = 2
NSUB = 16
NW = NCORES * NSUB
EPT = EDGES // NW
CH = 128
RPT = NPAD // NSUB

_GATHER_MODE = lax.GatherScatterMode.PROMISE_IN_BOUNDS


_GATHER_DNUMS = lax.GatherDimensionNumbers(
    offset_dims=(), collapsed_slice_dims=(0,), start_index_map=(0,))


def _take16(vec, idx):
    return lax.gather(vec, idx.reshape(16, 1), _GATHER_DNUMS, (1,),
                      mode=_GATHER_MODE)


def _iota16():
    return lax.iota(jnp.int32, 16)


_BM = 512
_GRID = NPAD // _BM


def _tc1_body(x_ref, w_ref, g_ref, gd_ref, hs_ref, ad_ref):
    h = jnp.dot(x_ref[...], w_ref[...], preferred_element_type=F32)
    hs_ref[...] = jnp.dot(h, g_ref[...], preferred_element_type=F32)
    ad_ref[...] = jnp.dot(h, gd_ref[...], preferred_element_type=F32)


def _tc1(xp, W, G, Gd):
    return pl.pallas_call(
        _tc1_body,
        grid=(_GRID,),
        in_specs=[
            pl.BlockSpec((_BM, xp.shape[1]), lambda i: (i, 0)),
            pl.BlockSpec(W.shape, lambda i: (0, 0)),
            pl.BlockSpec(G.shape, lambda i: (0, 0)),
            pl.BlockSpec(Gd.shape, lambda i: (0, 0)),
        ],
        out_specs=[
            pl.BlockSpec((_BM, 80), lambda i: (i, 0)),
            pl.BlockSpec((_BM, 16), lambda i: (i, 0)),
        ],
        out_shape=[
            jax.ShapeDtypeStruct((NPAD, 80), F32),
            jax.ShapeDtypeStruct((NPAD, 16), F32),
        ],
    )(xp, W, G, Gd)


def _act_from_parts(p_ref, b_ref, er_ref):
    psum = p_ref[0] + p_ref[1]
    num = psum[:, 0:64]
    den8 = psum[:, 64:72]
    den = jnp.dot(den8, er_ref[...], preferred_element_type=F32)
    act = num / (den + 1e-16) + b_ref[0][None, :]
    return jnp.maximum(act, 0.2 * act)


def _tcmid_body(p_ref, b_ref, er_ref, w_ref, g_ref, gd_ref, hs_ref, ad_ref):
    act = _act_from_parts(p_ref, b_ref, er_ref)
    h = jnp.dot(act, w_ref[...], preferred_element_type=F32)
    hs_ref[...] = jnp.dot(h, g_ref[...], preferred_element_type=F32)
    ad_ref[...] = jnp.dot(h, gd_ref[...], preferred_element_type=F32)


def _tcmid(p, b_cm, Erep, W_cm, G, Gd):
    return pl.pallas_call(
        _tcmid_body,
        grid=(_GRID,),
        in_specs=[
            pl.BlockSpec((2, _BM, 80), lambda i: (0, i, 0)),
            pl.BlockSpec((1, 64), lambda i: (0, 0)),
            pl.BlockSpec((8, 64), lambda i: (0, 0)),
            pl.BlockSpec((64, 64), lambda i: (0, 0)),
            pl.BlockSpec((64, 80), lambda i: (0, 0)),
            pl.BlockSpec((64, 16), lambda i: (0, 0)),
        ],
        out_specs=[
            pl.BlockSpec((_BM, 80), lambda i: (i, 0)),
            pl.BlockSpec((_BM, 16), lambda i: (i, 0)),
        ],
        out_shape=[
            jax.ShapeDtypeStruct((NPAD, 80), F32),
            jax.ShapeDtypeStruct((NPAD, 16), F32),
        ],
    )(p, b_cm, Erep, W_cm, G, Gd)


def _tcpre5_body(p_ref, b_ref, er_ref, w_ref, ga_ref, gd_ref,
                 hs_ref, ad_ref, as_ref):
    act = _act_from_parts(p_ref, b_ref, er_ref)
    h5 = jnp.dot(act, w_ref[...], preferred_element_type=F32)
    asd = jnp.dot(h5, ga_ref[...], preferred_element_type=F32)
    add = jnp.dot(h5, gd_ref[...], preferred_element_type=F32)
    hs_ref[:, 0:256] = h5
    hs_ref[:, 256:272] = asd
    as_ref[...] = asd
    ad_ref[...] = add


def _tcpre5(p, b_cm, Erep, W5_cm, G5a, G5d):
    return pl.pallas_call(
        _tcpre5_body,
        grid=(_GRID,),
        in_specs=[
            pl.BlockSpec((2, _BM, 80), lambda i: (0, i, 0)),
            pl.BlockSpec((1, 64), lambda i: (0, 0)),
            pl.BlockSpec((8, 64), lambda i: (0, 0)),
            pl.BlockSpec((64, 256), lambda i: (0, 0)),
            pl.BlockSpec((256, 16), lambda i: (0, 0)),
            pl.BlockSpec((256, 16), lambda i: (0, 0)),
        ],
        out_specs=[
            pl.BlockSpec((_BM, 272), lambda i: (i, 0)),
            pl.BlockSpec((_BM, 16), lambda i: (i, 0)),
            pl.BlockSpec((_BM, 16), lambda i: (i, 0)),
        ],
        out_shape=[
            jax.ShapeDtypeStruct((NPAD, 272), F32),
            jax.ShapeDtypeStruct((NPAD, 16), F32),
            jax.ShapeDtypeStruct((NPAD, 16), F32),
        ],
    )(p, b_cm, Erep, W5_cm, G5a, G5d)


def _tcmid5_body(ad_ref, da_ref, adr_ref):
    dsum = da_ref[0] + da_ref[1]
    rcp = 1.0 / (dsum + 1e-16)
    adr_ref[:, 0:8] = ad_ref[:, 0:8]
    adr_ref[:, 8:16] = rcp[:, 0:8]


def _tcmid5(ad5, dA):
    return pl.pallas_call(
        _tcmid5_body,
        grid=(_GRID,),
        in_specs=[
            pl.BlockSpec((_BM, 16), lambda i: (i, 0)),
            pl.BlockSpec((2, _BM, 16), lambda i: (0, i, 0)),
        ],
        out_specs=[pl.BlockSpec((_BM, 16), lambda i: (i, 0))],
        out_shape=[jax.ShapeDtypeStruct((NPAD, 16), F32)],
    )(ad5, dA)[0]


def _tcfinal_body(q_ref, b_ref, o_ref):
    logits = (q_ref[0] + q_ref[1]) * 0.125 + b_ref[0][None, :]
    m = jnp.max(logits, axis=1, keepdims=True)
    z = logits - m
    lse = jnp.log(jnp.sum(jnp.exp(z), axis=1, keepdims=True))
    o_ref[...] = z - lse


def _tcfinal(q, b5):
    return pl.pallas_call(
        _tcfinal_body,
        grid=(_GRID,),
        in_specs=[
            pl.BlockSpec((2, _BM, NCLS), lambda i: (0, i, 0)),
            pl.BlockSpec((1, NCLS), lambda i: (0, 0)),
        ],
        out_specs=[pl.BlockSpec((_BM, NCLS), lambda i: (i, 0))],
        out_shape=[jax.ShapeDtypeStruct((NPAD, NCLS), F32)],
    )(q, b5.reshape(1, NCLS))[0]


_MESH = plsc.VectorSubcoreMesh(core_axis_name="c", subcore_axis_name="s")
_SC_PARAMS = pltpu.CompilerParams(use_tc_tiling_on_sc=False)

_TCH = EDGES // CH
_CBASE = _TCH // NW
_CEXTRA = _TCH % NW


def _make_sc_body(edge_fn):
    def body(t_hbm, d_hbm, srce_hbm, dste_hbm, z_hbm, out_hbm, acc,
             s0, dd0, s1, dd1, t0, t1, r0, r1, crows,
             st0, st1, sd0, sd1):
        c = lax.axis_index("c")
        s = lax.axis_index("s")
        wid = s * NCORES + c
        lanes = _iota16()

        pltpu.sync_copy(z_hbm, acc.at[pl.ds(s * RPT, RPT)])
        plsc.subcore_barrier()

        nck = _CBASE + jnp.where(wid < _CEXTRA, 1, 0)
        srcs = (s0, s1)
        dsts = (dd0, dd1)
        trows = (t0, t1)
        drows = (r0, r1)
        sems_t = (st0, st1)
        sems_d = (sd0, sd1)

        def issue(k, b):
            off = (wid + NW * k) * CH
            pltpu.sync_copy(srce_hbm.at[pl.ds(off, CH)], srcs[b])
            pltpu.sync_copy(dste_hbm.at[pl.ds(off, CH)], dsts[b])
            pltpu.async_copy(t_hbm.at[srcs[b]], trows[b], sems_t[b])
            pltpu.async_copy(d_hbm.at[dsts[b]], drows[b], sems_d[b])

        def wait(b):
            pltpu.make_async_copy(t_hbm.at[srcs[b]], trows[b], sems_t[b]).wait()
            pltpu.make_async_copy(d_hbm.at[dsts[b]], drows[b], sems_d[b]).wait()

        @pl.when(nck > 0)
        def _():
            issue(0, 0)

        @pl.when(nck > 1)
        def _():
            issue(1, 1)

        def step(j, carry):
            for b in (0, 1):
                k = 2 * j + b

                @pl.when(k < nck)
                def _():
                    wait(b)

                    def edge(e, cc):
                        edge_fn(e, trows[b], drows[b], crows, lanes)
                        return cc

                    lax.fori_loop(0, CH, edge, 0, unroll=4)
                    pltpu.sync_copy(crows, acc.at[dsts[b]], add=True)

                    @pl.when(k + 2 < nck)
                    def _():
                        issue(k + 2, b)

            return carry

        lax.fori_loop(0, (_CBASE + 2) // 2, step, 0)
        plsc.subcore_barrier()
        pltpu.sync_copy(acc.at[pl.ds(s * RPT, RPT)],
                        out_hbm.at[c, pl.ds(s * RPT, RPT)])

    return body


def _sc_scratch(tw, dw, ow):
    return [
        pltpu.VMEM_SHARED((NPAD, ow), F32),
        pltpu.VMEM((CH,), jnp.int32),
        pltpu.VMEM((CH,), jnp.int32),
        pltpu.VMEM((CH,), jnp.int32),
        pltpu.VMEM((CH,), jnp.int32),
        pltpu.VMEM((CH, tw), F32),
        pltpu.VMEM((CH, tw), F32),
        pltpu.VMEM((CH, dw), F32),
        pltpu.VMEM((CH, dw), F32),
        pltpu.VMEM((CH, ow), F32),
        pltpu.SemaphoreType.DMA,
        pltpu.SemaphoreType.DMA,
        pltpu.SemaphoreType.DMA,
        pltpu.SemaphoreType.DMA,
    ]


def _edge_layer(e, trow, drow, crows, lanes):
    idxrep = lanes % 8
    a_s = trow[e, pl.ds(64, 16)]
    a_d = drow[e, pl.ds(0, 16)]
    t = a_s + a_d
    t = jnp.maximum(t, 0.2 * t)
    ex = jnp.exp(t)
    w = _take16(ex, idxrep)
    crows[e, pl.ds(0, 16)] = trow[e, pl.ds(0, 16)] * w
    crows[e, pl.ds(16, 16)] = trow[e, pl.ds(16, 16)] * w
    crows[e, pl.ds(32, 16)] = trow[e, pl.ds(32, 16)] * w
    crows[e, pl.ds(48, 16)] = trow[e, pl.ds(48, 16)] * w
    crows[e, pl.ds(64, 16)] = jnp.where(lanes < 8, ex, 0.0)


def _edge_denom5(e, trow, drow, crows, lanes):
    t = trow[e, pl.ds(0, 16)] + drow[e, pl.ds(0, 16)]
    t = jnp.maximum(t, 0.2 * t)
    crows[e, pl.ds(0, 16)] = jnp.where(lanes < 8, jnp.exp(t), 0.0)


def _edge_aggr5(e, trow, drow, crows, lanes):
    idxrcp = lanes % 8 + 8
    a_s = trow[e, pl.ds(256, 16)]
    adr = drow[e, pl.ds(0, 16)]
    t = a_s + adr
    t = jnp.maximum(t, 0.2 * t)
    ex = jnp.exp(t)
    w = ex * _take16(adr, idxrcp)
    acc0 = jnp.zeros((16,), F32)
    acc1 = jnp.zeros((16,), F32)
    for kk in range(16):
        wk = _take16(w, jnp.full((16,), kk // 2, jnp.int32))
        hk = trow[e, pl.ds(16 * kk, 16)]
        if kk % 2 == 0:
            acc0 = acc0 + wk * hk
        else:
            acc1 = acc1 + wk * hk
    crows[e, pl.ds(0, 16)] = acc0
    crows[e, pl.ds(16, 16)] = acc1


def _sc_layer(hs, ad, srce, dste, z80):
    f = pl.kernel(
        _make_sc_body(_edge_layer),
        compiler_params=_SC_PARAMS,
        out_type=jax.ShapeDtypeStruct((2, NPAD, 80), F32),
        mesh=_MESH,
        scratch_types=_sc_scratch(80, 16, 80),
    )
    return f(hs, ad, srce, dste, z80)


def _sc_denom5(as5, ad5, srce, dste, z16):
    f = pl.kernel(
        _make_sc_body(_edge_denom5),
        compiler_params=_SC_PARAMS,
        out_type=jax.ShapeDtypeStruct((2, NPAD, 16), F32),
        mesh=_MESH,
        scratch_types=_sc_scratch(16, 16, 16),
    )
    return f(as5, ad5, srce, dste, z16)


def _sc_aggr5(hs5, adr, srce, dste, z32):
    f = pl.kernel(
        _make_sc_body(_edge_aggr5),
        compiler_params=_SC_PARAMS,
        out_type=jax.ShapeDtypeStruct((2, NPAD, NCLS), F32),
        mesh=_MESH,
        scratch_types=_sc_scratch(272, 16, NCLS),
    )
    return f(hs5, adr, srce, dste, z32)


def _sel_mats(a_src, a_dst):
    hh, cc = a_src.shape
    eye = jnp.eye(hh, dtype=F32)
    s_src = jnp.einsum("hc,hk->hck", a_src, eye).reshape(hh * cc, hh)
    s_dst = jnp.einsum("hc,hk->hck", a_dst, eye).reshape(hh * cc, hh)
    return s_src, s_dst


def _perm64():
    ar = jnp.arange(64)
    perm = (ar % 8) * 8 + ar // 8
    return jnp.eye(64, dtype=F32)[perm]


def kernel(x, edge_index, W1, a_src1, a_dst1, b1, W2, a_src2, a_dst2, b2,
           W3, a_src3, a_dst3, b3, W4, a_src4, a_dst4, b4,
           W5, a_src5, a_dst5, b5):
    P = _perm64()
    Erep = jnp.tile(jnp.eye(8, dtype=F32), (1, 8))
    z8 = jnp.zeros((64, 8), F32)
    z80 = jnp.zeros((RPT, 80), F32)
    z16 = jnp.zeros((RPT, 16), F32)
    z32 = jnp.zeros((RPT, NCLS), F32)

    def build_G(a_src, a_dst):
        s_src, s_dst = _sel_mats(a_src, a_dst)
        G = jnp.concatenate([P, s_src, z8], axis=1)
        Gd = jnp.concatenate([s_dst, z8], axis=1)
        return G, Gd

    xp = jnp.pad(x, ((0, NPAD - NN), (0, 0)))
    ei = edge_index.astype(jnp.int32)
    srce, dste = ei[0], ei[1]

    G, Gd = build_G(a_src1, a_dst1)
    hs, ad = _tc1(xp, W1, G, Gd)
    p = _sc_layer(hs, ad, srce, dste, z80)

    for W, a_s_, a_d_, b_prev in (
        (W2, a_src2, a_dst2, b1),
        (W3, a_src3, a_dst3, b2),
        (W4, a_src4, a_dst4, b3),
    ):
        G, Gd = build_G(a_s_, a_d_)
        hs, ad = _tcmid(p, (b_prev @ P).reshape(1, 64), Erep, P @ W, G, Gd)
        p = _sc_layer(hs, ad, srce, dste, z80)

    s5_src, s5_dst = _sel_mats(a_src5, a_dst5)
    z58 = jnp.zeros((256, 8), F32)
    G5a = jnp.concatenate([s5_src, z58], axis=1)
    G5d = jnp.concatenate([s5_dst, z58], axis=1)
    hs5, ad5, as5 = _tcpre5(p, (b4 @ P).reshape(1, 64), Erep, P @ W5, G5a, G5d)
    dA = _sc_denom5(as5, ad5, srce, dste, z16)
    adr = _tcmid5(ad5, dA)
    q = _sc_aggr5(hs5, adr, srce, dste, z32)
    out = _tcfinal(q, b5)
    return out[:NN]


# speedup vs baseline: 84.1610x; 1.0668x over previous
import functools

import jax
import jax.numpy as jnp
from jax import lax
from jax.experimental import pallas as pl
from jax.experimental.pallas import tpu as pltpu
from jax.experimental.pallas import tpu_sc as plsc

NN = 10000
NPAD = 10240
EDGES = 320000
NCLS = 32
F32 = jnp.float32

NCORES = 2
NSUB = 16
NW = NCORES * NSUB
CH = 128
EPAD = 327680
RPT = NPAD // NSUB

_GATHER_MODE = lax.GatherScatterMode.PROMISE_IN_BOUNDS


_GATHER_DNUMS = lax.GatherDimensionNumbers(
    offset_dims=(), collapsed_slice_dims=(0,), start_index_map=(0,))


def _take16(vec, idx):
    return lax.gather(vec, idx.reshape(16, 1), _GATHER_DNUMS, (1,),
                      mode=_GATHER_MODE)


def _iota16():
    return lax.iota(jnp.int32, 16)


_BM = 512
_GRID = NPAD // _BM


def _tc1_body(x_ref, w_ref, g_ref, gd_ref, hs_ref, ad_ref):
    h = jnp.dot(x_ref[...], w_ref[...], preferred_element_type=F32)
    hs_ref[...] = jnp.dot(h, g_ref[...], preferred_element_type=F32)
    ad_ref[...] = jnp.dot(h, gd_ref[...], preferred_element_type=F32)


def _tc1(xp, W, G, Gd):
    return pl.pallas_call(
        _tc1_body,
        grid=(_GRID,),
        in_specs=[
            pl.BlockSpec((_BM, xp.shape[1]), lambda i: (i, 0)),
            pl.BlockSpec(W.shape, lambda i: (0, 0)),
            pl.BlockSpec(G.shape, lambda i: (0, 0)),
            pl.BlockSpec(Gd.shape, lambda i: (0, 0)),
        ],
        out_specs=[
            pl.BlockSpec((_BM, 80), lambda i: (i, 0)),
            pl.BlockSpec((_BM, 16), lambda i: (i, 0)),
        ],
        out_shape=[
            jax.ShapeDtypeStruct((NPAD, 80), F32),
            jax.ShapeDtypeStruct((NPAD, 16), F32),
        ],
    )(xp, W, G, Gd)


def _act_from_parts(p_ref, b_ref, er_ref):
    psum = p_ref[0] + p_ref[1]
    num = psum[:, 0:64]
    den8 = psum[:, 64:72]
    den = jnp.dot(den8, er_ref[...], preferred_element_type=F32)
    act = num / (den + 1e-16) + b_ref[0][None, :]
    return jnp.maximum(act, 0.2 * act)


def _tcmid_body(p_ref, b_ref, er_ref, w_ref, g_ref, gd_ref, hs_ref, ad_ref):
    act = _act_from_parts(p_ref, b_ref, er_ref)
    h = jnp.dot(act, w_ref[...], preferred_element_type=F32)
    hs_ref[...] = jnp.dot(h, g_ref[...], preferred_element_type=F32)
    ad_ref[...] = jnp.dot(h, gd_ref[...], preferred_element_type=F32)


def _tcmid(p, b_cm, Erep, W_cm, G, Gd):
    return pl.pallas_call(
        _tcmid_body,
        grid=(_GRID,),
        in_specs=[
            pl.BlockSpec((2, _BM, 80), lambda i: (0, i, 0)),
            pl.BlockSpec((1, 64), lambda i: (0, 0)),
            pl.BlockSpec((8, 64), lambda i: (0, 0)),
            pl.BlockSpec((64, 64), lambda i: (0, 0)),
            pl.BlockSpec((64, 80), lambda i: (0, 0)),
            pl.BlockSpec((64, 16), lambda i: (0, 0)),
        ],
        out_specs=[
            pl.BlockSpec((_BM, 80), lambda i: (i, 0)),
            pl.BlockSpec((_BM, 16), lambda i: (i, 0)),
        ],
        out_shape=[
            jax.ShapeDtypeStruct((NPAD, 80), F32),
            jax.ShapeDtypeStruct((NPAD, 16), F32),
        ],
    )(p, b_cm, Erep, W_cm, G, Gd)


def _tcpre5_body(p_ref, b_ref, er_ref, w_ref, ga_ref, gd_ref,
                 hs_ref, ad_ref, as_ref):
    act = _act_from_parts(p_ref, b_ref, er_ref)
    h5 = jnp.dot(act, w_ref[...], preferred_element_type=F32)
    asd = jnp.dot(h5, ga_ref[...], preferred_element_type=F32)
    add = jnp.dot(h5, gd_ref[...], preferred_element_type=F32)
    hs_ref[:, 0:256] = h5
    hs_ref[:, 256:272] = asd
    as_ref[...] = asd
    ad_ref[...] = add


def _tcpre5(p, b_cm, Erep, W5_cm, G5a, G5d):
    return pl.pallas_call(
        _tcpre5_body,
        grid=(_GRID,),
        in_specs=[
            pl.BlockSpec((2, _BM, 80), lambda i: (0, i, 0)),
            pl.BlockSpec((1, 64), lambda i: (0, 0)),
            pl.BlockSpec((8, 64), lambda i: (0, 0)),
            pl.BlockSpec((64, 256), lambda i: (0, 0)),
            pl.BlockSpec((256, 16), lambda i: (0, 0)),
            pl.BlockSpec((256, 16), lambda i: (0, 0)),
        ],
        out_specs=[
            pl.BlockSpec((_BM, 272), lambda i: (i, 0)),
            pl.BlockSpec((_BM, 16), lambda i: (i, 0)),
            pl.BlockSpec((_BM, 16), lambda i: (i, 0)),
        ],
        out_shape=[
            jax.ShapeDtypeStruct((NPAD, 272), F32),
            jax.ShapeDtypeStruct((NPAD, 16), F32),
            jax.ShapeDtypeStruct((NPAD, 16), F32),
        ],
    )(p, b_cm, Erep, W5_cm, G5a, G5d)


def _tcmid5_body(ad_ref, da_ref, adr_ref):
    dsum = da_ref[0] + da_ref[1]
    rcp = 1.0 / (dsum + 1e-16)
    adr_ref[:, 0:8] = ad_ref[:, 0:8]
    adr_ref[:, 8:16] = rcp[:, 0:8]


def _tcmid5(ad5, dA):
    return pl.pallas_call(
        _tcmid5_body,
        grid=(_GRID,),
        in_specs=[
            pl.BlockSpec((_BM, 16), lambda i: (i, 0)),
            pl.BlockSpec((2, _BM, 16), lambda i: (0, i, 0)),
        ],
        out_specs=[pl.BlockSpec((_BM, 16), lambda i: (i, 0))],
        out_shape=[jax.ShapeDtypeStruct((NPAD, 16), F32)],
    )(ad5, dA)[0]


def _tcfinal_body(q_ref, b_ref, o_ref):
    logits = (q_ref[0] + q_ref[1]) * 0.125 + b_ref[0][None, :]
    m = jnp.max(logits, axis=1, keepdims=True)
    z = logits - m
    lse = jnp.log(jnp.sum(jnp.exp(z), axis=1, keepdims=True))
    o_ref[...] = z - lse


def _tcfinal(q, b5):
    return pl.pallas_call(
        _tcfinal_body,
        grid=(_GRID,),
        in_specs=[
            pl.BlockSpec((2, _BM, NCLS), lambda i: (0, i, 0)),
            pl.BlockSpec((1, NCLS), lambda i: (0, 0)),
        ],
        out_specs=[pl.BlockSpec((_BM, NCLS), lambda i: (i, 0))],
        out_shape=[jax.ShapeDtypeStruct((NPAD, NCLS), F32)],
    )(q, b5.reshape(1, NCLS))[0]


_MESH = plsc.VectorSubcoreMesh(core_axis_name="c", subcore_axis_name="s")
_SC_PARAMS = pltpu.CompilerParams(use_tc_tiling_on_sc=False)

IB = 8
NCK = EPAD // NW // CH
NB = NCK // IB
CROWS_HBM = EPAD // CH


def _make_sc_body(edge_fn, nbuf_g):
    assert IB % nbuf_g == 0

    def body(t_hbm, d_hbm, srcr_hbm, dstr_hbm, z_hbm, out_hbm, acc,
             si0, si1, di0, di1, trows, drows, crows0, crows1,
             semg, sems0, sems1):
        c = lax.axis_index("c")
        s = lax.axis_index("s")
        wid = s * NCORES + c
        rowbase = wid * NCK
        lanes = _iota16()

        pltpu.sync_copy(z_hbm, acc.at[pl.ds(s * RPT, RPT)])
        plsc.subcore_barrier()

        sidx = (si0, si1)
        didx = (di0, di1)
        crows = (crows0, crows1)
        sems = (sems0, sems1)

        def load_idx(jb, pb):
            pltpu.sync_copy(srcr_hbm.at[pl.ds(rowbase + IB * jb, IB)], sidx[pb])
            pltpu.sync_copy(dstr_hbm.at[pl.ds(rowbase + IB * jb, IB)], didx[pb])

        def issue_g(i, jb_parity, g):
            pltpu.async_copy(t_hbm.at[sidx[jb_parity].at[i]], trows[g], semg[g])
            pltpu.async_copy(d_hbm.at[didx[jb_parity].at[i]], drows[g], semg[g])

        def wait_g(i, jb_parity, g):
            pltpu.make_async_copy(t_hbm.at[sidx[jb_parity].at[i]], trows[g],
                                  semg[g]).wait()
            pltpu.make_async_copy(d_hbm.at[didx[jb_parity].at[i]], drows[g],
                                  semg[g]).wait()

        def issue_s(i, jb_parity, cb):
            pltpu.async_copy(crows[cb], acc.at[didx[jb_parity].at[i]],
                             sems[cb], add=True)

        def wait_s(i, jb_parity, cb):
            pltpu.make_async_copy(crows[cb], acc.at[didx[jb_parity].at[i]],
                                  sems[cb]).wait()

        load_idx(0, 0)
        for i in range(nbuf_g):
            issue_g(i, 0, i)

        def batchpair(jj, carry):
          for pb in (0, 1):
            jb = 2 * jj + pb
            for i in range(IB):
                g = i % nbuf_g
                cb = i % 2
                k = jb * IB + i
                wait_g(i, pb, g)

                if i >= 2:
                    wait_s(i - 2, pb, cb)
                else:
                    @pl.when(jb > 0)
                    def _():
                        wait_s(IB - 2 + i, (pb + 1) % 2, cb)

                def edge(e, cc):
                    edge_fn(e, trows[g], drows[g], crows[cb], lanes)
                    return cc

                lax.fori_loop(0, CH, edge, 0, unroll=4)
                issue_s(i, pb, cb)

                if i == 2:
                    @pl.when(jb + 1 < NB)
                    def _():
                        load_idx(jb + 1, (pb + 1) % 2)

                ii = i + nbuf_g
                if ii < IB:
                    issue_g(ii, pb, g)
                else:
                    @pl.when(jb + 1 < NB)
                    def _():
                        issue_g(ii - IB, (pb + 1) % 2, g)

          return carry

        lax.fori_loop(0, NB // 2, batchpair, 0)
        wait_s(IB - 2, (NB - 1) % 2, 0)
        wait_s(IB - 1, (NB - 1) % 2, 1)
        plsc.subcore_barrier()
        pltpu.sync_copy(acc.at[pl.ds(s * RPT, RPT)],
                        out_hbm.at[c, pl.ds(s * RPT, RPT)])

    return body


def _sc_scratch(tw, dw, ow, nbuf_g):
    return [
        pltpu.VMEM_SHARED((NPAD, ow), F32),
        pltpu.VMEM((IB, CH), jnp.int32),
        pltpu.VMEM((IB, CH), jnp.int32),
        pltpu.VMEM((IB, CH), jnp.int32),
        pltpu.VMEM((IB, CH), jnp.int32),
        [pltpu.VMEM((CH, tw), F32) for _ in range(nbuf_g)],
        [pltpu.VMEM((CH, dw), F32) for _ in range(nbuf_g)],
        pltpu.VMEM((CH, ow), F32),
        pltpu.VMEM((CH, ow), F32),
        [pltpu.SemaphoreType.DMA for _ in range(nbuf_g)],
        pltpu.SemaphoreType.DMA,
        pltpu.SemaphoreType.DMA,
    ]


def _edge_layer(e, trow, drow, crows, lanes):
    idxrep = lanes % 8
    a_s = trow[e, pl.ds(64, 16)]
    a_d = drow[e, pl.ds(0, 16)]
    t = a_s + a_d
    t = jnp.maximum(t, 0.2 * t)
    ex = jnp.exp(t)
    w = _take16(ex, idxrep)
    crows[e, pl.ds(0, 16)] = trow[e, pl.ds(0, 16)] * w
    crows[e, pl.ds(16, 16)] = trow[e, pl.ds(16, 16)] * w
    crows[e, pl.ds(32, 16)] = trow[e, pl.ds(32, 16)] * w
    crows[e, pl.ds(48, 16)] = trow[e, pl.ds(48, 16)] * w
    crows[e, pl.ds(64, 16)] = jnp.where(lanes < 8, ex, 0.0)


def _edge_denom5(e, trow, drow, crows, lanes):
    t = trow[e, pl.ds(0, 16)] + drow[e, pl.ds(0, 16)]
    t = jnp.maximum(t, 0.2 * t)
    crows[e, pl.ds(0, 16)] = jnp.where(lanes < 8, jnp.exp(t), 0.0)


def _edge_aggr5(e, trow, drow, crows, lanes):
    idxrcp = lanes % 8 + 8
    a_s = trow[e, pl.ds(256, 16)]
    adr = drow[e, pl.ds(0, 16)]
    t = a_s + adr
    t = jnp.maximum(t, 0.2 * t)
    ex = jnp.exp(t)
    w = ex * _take16(adr, idxrcp)
    acc0 = jnp.zeros((16,), F32)
    acc1 = jnp.zeros((16,), F32)
    for kk in range(16):
        wk = _take16(w, jnp.full((16,), kk // 2, jnp.int32))
        hk = trow[e, pl.ds(16 * kk, 16)]
        if kk % 2 == 0:
            acc0 = acc0 + wk * hk
        else:
            acc1 = acc1 + wk * hk
    crows[e, pl.ds(0, 16)] = acc0
    crows[e, pl.ds(16, 16)] = acc1


def _sc_layer(hs, ad, srcr, dstr, z80):
    f = pl.kernel(
        _make_sc_body(_edge_layer, 4),
        compiler_params=_SC_PARAMS,
        out_type=jax.ShapeDtypeStruct((2, NPAD, 80), F32),
        mesh=_MESH,
        scratch_types=_sc_scratch(80, 16, 80, 4),
    )
    return f(hs, ad, srcr, dstr, z80)


def _sc_denom5(as5, ad5, srcr, dstr, z16):
    f = pl.kernel(
        _make_sc_body(_edge_denom5, 4),
        compiler_params=_SC_PARAMS,
        out_type=jax.ShapeDtypeStruct((2, NPAD, 16), F32),
        mesh=_MESH,
        scratch_types=_sc_scratch(16, 16, 16, 4),
    )
    return f(as5, ad5, srcr, dstr, z16)


def _sc_aggr5(hs5, adr, srcr, dstr, z32):
    f = pl.kernel(
        _make_sc_body(_edge_aggr5, 2),
        compiler_params=_SC_PARAMS,
        out_type=jax.ShapeDtypeStruct((2, NPAD, NCLS), F32),
        mesh=_MESH,
        scratch_types=_sc_scratch(272, 16, NCLS, 2),
    )
    return f(hs5, adr, srcr, dstr, z32)


def _sel_mats(a_src, a_dst):
    hh, cc = a_src.shape
    eye = jnp.eye(hh, dtype=F32)
    s_src = jnp.einsum("hc,hk->hck", a_src, eye).reshape(hh * cc, hh)
    s_dst = jnp.einsum("hc,hk->hck", a_dst, eye).reshape(hh * cc, hh)
    return s_src, s_dst


def _perm64():
    ar = jnp.arange(64)
    perm = (ar % 8) * 8 + ar // 8
    return jnp.eye(64, dtype=F32)[perm]


def kernel(x, edge_index, W1, a_src1, a_dst1, b1, W2, a_src2, a_dst2, b2,
           W3, a_src3, a_dst3, b3, W4, a_src4, a_dst4, b4,
           W5, a_src5, a_dst5, b5):
    P = _perm64()
    Erep = jnp.tile(jnp.eye(8, dtype=F32), (1, 8))
    z8 = jnp.zeros((64, 8), F32)
    z80 = jnp.zeros((RPT, 80), F32)
    z16 = jnp.zeros((RPT, 16), F32)
    z32 = jnp.zeros((RPT, NCLS), F32)

    def build_G(a_src, a_dst):
        s_src, s_dst = _sel_mats(a_src, a_dst)
        G = jnp.concatenate([P, s_src, z8], axis=1)
        Gd = jnp.concatenate([s_dst, z8], axis=1)
        return G, Gd

    xp = jnp.pad(x, ((0, NPAD - NN), (0, 0)))
    ei = edge_index.astype(jnp.int32)
    srce = jnp.pad(ei[0], (0, EPAD - EDGES),
                   constant_values=NPAD - 1).reshape(EPAD // CH, CH)
    dste = jnp.pad(ei[1], (0, EPAD - EDGES),
                   constant_values=NPAD - 1).reshape(EPAD // CH, CH)

    G, Gd = build_G(a_src1, a_dst1)
    hs, ad = _tc1(xp, W1, G, Gd)
    p = _sc_layer(hs, ad, srce, dste, z80)

    for W, a_s_, a_d_, b_prev in (
        (W2, a_src2, a_dst2, b1),
        (W3, a_src3, a_dst3, b2),
        (W4, a_src4, a_dst4, b3),
    ):
        G, Gd = build_G(a_s_, a_d_)
        hs, ad = _tcmid(p, (b_prev @ P).reshape(1, 64), Erep, P @ W, G, Gd)
        p = _sc_layer(hs, ad, srce, dste, z80)

    s5_src, s5_dst = _sel_mats(a_src5, a_dst5)
    z58 = jnp.zeros((256, 8), F32)
    G5a = jnp.concatenate([s5_src, z58], axis=1)
    G5d = jnp.concatenate([s5_dst, z58], axis=1)
    hs5, ad5, as5 = _tcpre5(p, (b4 @ P).reshape(1, 64), Erep, P @ W5, G5a, G5d)
    dA = _sc_denom5(as5, ad5, srce, dste, z16)
    adr = _tcmid5(ad5, dA)
    q = _sc_aggr5(hs5, adr, srce, dste, z32)
    out = _tcfinal(q, b5)
    return out[:NN]
